# baseline (speedup 1.0000x reference)
import functools

import jax
import jax.numpy as jnp
from jax import lax
from jax.experimental import pallas as pl
from jax.experimental.pallas import tpu as pltpu
from jax.experimental.pallas import tpu_sc as plsc

N = 10000
E = 320000
D = 128

NC = 2
NS = 16
NW = NC * NS

CHUNK = 128
CPT = 80
NCHUNKS = NW * CPT
E_PAD = NCHUNKS * CHUNK
NP_ = 10240
PAD_ROW = N
RPS = NP_ // NS

_MESH = plsc.VectorSubcoreMesh(core_axis_name="c", subcore_axis_name="s")


def _scalar_agg_body(gather, vals_hbm, src_hbm, dst_hbm, zeros_hbm, out_hbm,
                     sivb, divb, vbuf, acc, vsh, gsem, ssem):
    cid = lax.axis_index("c")
    sid = lax.axis_index("s")
    wid = sid * NC + cid
    c0 = wid * CPT
    r0 = sid * RPS
    pltpu.sync_copy(zeros_hbm.at[pl.ds(r0, RPS)], acc.at[pl.ds(r0, RPS)])
    pltpu.sync_copy(dst_hbm.at[pl.ds(c0, CPT)], divb)
    if gather:
        pltpu.sync_copy(src_hbm.at[pl.ds(c0, CPT)], sivb)
        pltpu.sync_copy(vals_hbm.at[pl.ds(r0, RPS)], vsh.at[pl.ds(r0, RPS)])
    else:
        for j in range(CHUNK // 16):
            vbuf[0, pl.ds(j * 16, 16)] = jnp.full((16,), 1.0, jnp.float32)
    plsc.subcore_barrier()

    def drain_g(b):
        pltpu.make_async_copy(vsh.at[sivb.at[0]], vbuf.at[b], gsem).wait()

    def drain_s(b):
        pltpu.make_async_copy(vbuf.at[b], acc.at[divb.at[0]], ssem).wait()

    if gather:
        pltpu.async_copy(vsh.at[sivb.at[0]], vbuf.at[0], gsem)

        def pair(i, carry):
            for b in (0, 1):
                c = 2 * i + b
                drain_g(b)
                pltpu.async_copy(vbuf.at[b], acc.at[divb.at[c]], ssem, add=True)

                @pl.when(c + 1 < CPT)
                def _():
                    @pl.when(c >= 1)
                    def _():
                        drain_s(1 - b)
                    pltpu.async_copy(vsh.at[sivb.at[c + 1]], vbuf.at[1 - b],
                                     gsem)

            return carry

        lax.fori_loop(0, CPT // 2, pair, 0)
        drain_s(0)
        drain_s(1)
    else:
        def body(c, carry):
            @pl.when(c >= 8)
            def _():
                drain_s(0)
            pltpu.async_copy(vbuf.at[0], acc.at[divb.at[c]], ssem, add=True)
            return carry

        lax.fori_loop(0, CPT, body, 0)
        for _ in range(8):
            drain_s(0)
    plsc.subcore_barrier()
    pltpu.sync_copy(acc.at[pl.ds(r0, RPS)], out_hbm.at[cid].at[pl.ds(r0, RPS)])


def _make_scalar_agg(gather):
    return functools.partial(
        pl.kernel,
        out_type=jax.ShapeDtypeStruct((NC, NP_), jnp.float32),
        mesh=_MESH,
        scratch_types=[
            pltpu.VMEM((CPT, CHUNK), jnp.int32),
            pltpu.VMEM((CPT, CHUNK), jnp.int32),
            pltpu.VMEM((2, CHUNK), jnp.float32),
            pltpu.VMEM_SHARED((NP_,), jnp.float32),
            pltpu.VMEM_SHARED((NP_,), jnp.float32),
            pltpu.SemaphoreType.DMA,
            pltpu.SemaphoreType.DMA,
        ],
    )(functools.partial(_scalar_agg_body, gather))


_sc_scalar_agg = _make_scalar_agg(True)
_sc_degree = _make_scalar_agg(False)


@functools.partial(
    pl.kernel,
    out_type=jax.ShapeDtypeStruct((NC, NP_, D), jnp.float32),
    mesh=_MESH,
    scratch_types=[
        pltpu.VMEM((CPT // 2, CHUNK), jnp.int32),
        pltpu.VMEM((CPT // 2, CHUNK), jnp.int32),
        pltpu.VMEM((2, CHUNK, D), jnp.float32),
        pltpu.VMEM_SHARED((NP_, D), jnp.float32),
        pltpu.SemaphoreType.DMA,
        pltpu.SemaphoreType.DMA,
    ],
)
def _sc_dense_agg(hs_hbm, src_hbm, dst_hbm, zeros_hbm, out_hbm,
                  sivb, divb, rows, acc, gsem, ssem):
    cid = lax.axis_index("c")
    sid = lax.axis_index("s")
    wid = sid * NC + cid
    c0 = wid * CPT
    r0 = sid * RPS
    HC = CPT // 2
    pltpu.sync_copy(zeros_hbm.at[pl.ds(r0, RPS)], acc.at[pl.ds(r0, RPS)])
    plsc.subcore_barrier()

    def drain_g(b):
        pltpu.make_async_copy(hs_hbm.at[sivb.at[0]], rows.at[b], gsem).wait()

    def drain_s(b):
        pltpu.make_async_copy(rows.at[b], acc.at[divb.at[0]], ssem).wait()

    for p in (0, 1):
        pltpu.sync_copy(src_hbm.at[pl.ds(c0 + p * HC, HC)], sivb)
        pltpu.sync_copy(dst_hbm.at[pl.ds(c0 + p * HC, HC)], divb)
        pltpu.async_copy(hs_hbm.at[sivb.at[0]], rows.at[0], gsem)

        def pair(i, carry):
            for b in (0, 1):
                c = 2 * i + b
                drain_g(b)
                pltpu.async_copy(rows.at[b], acc.at[divb.at[c]], ssem,
                                 add=True)

                @pl.when(c + 1 < HC)
                def _():
                    @pl.when(c >= 1)
                    def _():
                        drain_s(1 - b)
                    pltpu.async_copy(hs_hbm.at[sivb.at[c + 1]], rows.at[1 - b],
                                     gsem)

            return carry

        lax.fori_loop(0, HC // 2, pair, 0)
        drain_s(0)
        drain_s(1)
    plsc.subcore_barrier()
    pltpu.sync_copy(acc.at[pl.ds(r0, RPS)], out_hbm.at[cid].at[pl.ds(r0, RPS)])


RB = 1000
GRID = N // RB


def _t0_body(x_ref, w_ref, h_ref):
    h_ref[...] = jnp.dot(x_ref[...], w_ref[...],
                         preferred_element_type=jnp.float32)


def _tc_matmul(x, W1):
    return pl.pallas_call(
        _t0_body,
        grid=(GRID,),
        in_specs=[
            pl.BlockSpec((RB, D), lambda i: (i, 0)),
            pl.BlockSpec((D, D), lambda i: (0, 0)),
        ],
        out_specs=pl.BlockSpec((RB, D), lambda i: (i, 0)),
        out_shape=jax.ShapeDtypeStruct((N, D), jnp.float32),
    )(x, W1)


def _t1_body(h_ref, dsum_ref, hs_ref, dinv_ref):
    dinv = lax.rsqrt(dsum_ref[...] + 1.0)
    hs_ref[...] = h_ref[...] * dinv
    dinv_ref[...] = dinv


def _tc_scale(h, degsum):
    return pl.pallas_call(
        _t1_body,
        grid=(GRID,),
        in_specs=[
            pl.BlockSpec((RB, D), lambda i: (i, 0)),
            pl.BlockSpec((RB, 1), lambda i: (i, 0)),
        ],
        out_specs=[
            pl.BlockSpec((RB, D), lambda i: (i, 0)),
            pl.BlockSpec((RB, 1), lambda i: (i, 0)),
        ],
        out_shape=[
            jax.ShapeDtypeStruct((N, D), jnp.float32),
            jax.ShapeDtypeStruct((N, 1), jnp.float32),
        ],
    )(h, degsum)


def _t2_body(a_ref, hs_ref, dinv_ref, b1_ref, w2_ref, s_ref):
    dinv = dinv_ref[...]
    o = (a_ref[0] + a_ref[1] + hs_ref[...]) * dinv + b1_ref[...]
    o = jnp.maximum(o, 0.0)
    s_ref[...] = jnp.dot(o, w2_ref[...], preferred_element_type=jnp.float32) * dinv


def _tc_post1(aggp, hs, dinv, b1, W2):
    return pl.pallas_call(
        _t2_body,
        grid=(GRID,),
        in_specs=[
            pl.BlockSpec((NC, RB, D), lambda i: (0, i, 0)),
            pl.BlockSpec((RB, D), lambda i: (i, 0)),
            pl.BlockSpec((RB, 1), lambda i: (i, 0)),
            pl.BlockSpec((1, D), lambda i: (0, 0)),
            pl.BlockSpec((D, 1), lambda i: (0, 0)),
        ],
        out_specs=pl.BlockSpec((RB, 1), lambda i: (i, 0)),
        out_shape=jax.ShapeDtypeStruct((N, 1), jnp.float32),
    )(aggp, hs, dinv, b1, W2)


def _t3_body(q0_ref, q1_ref, s_ref, dinv_ref, b2_ref, out_ref):
    pre = (q0_ref[...] + q1_ref[...] + s_ref[...]) * dinv_ref[...] + b2_ref[...]
    out_ref[...] = jax.nn.sigmoid(pre)


def _tc_post2(q0, q1, s, dinv, b2):
    return pl.pallas_call(
        _t3_body,
        out_shape=jax.ShapeDtypeStruct((N, 1), jnp.float32),
    )(q0, q1, s, dinv, b2)


def kernel(x, edge_index, W1, b1, W2, b2):
    npad = E_PAD - E
    src2d = jnp.concatenate(
        [edge_index[0].astype(jnp.int32),
         jnp.zeros((npad,), jnp.int32)]).reshape(NCHUNKS, CHUNK)
    pad_dst = PAD_ROW + jnp.arange(npad, dtype=jnp.int32) % (NP_ - N)
    dst2d = jnp.concatenate(
        [edge_index[1].astype(jnp.int32), pad_dst]).reshape(NCHUNKS, CHUNK)
    zeros1 = jnp.zeros((NP_,), jnp.float32)
    zeros2 = jnp.zeros((NP_, D), jnp.float32)

    h1 = _tc_matmul(x, W1)
    degp = _sc_degree(zeros1, src2d, dst2d, zeros1)
    degsum = (degp[0, :N] + degp[1, :N]).reshape(N, 1)
    hs1, dinv = _tc_scale(h1, degsum)
    aggp = _sc_dense_agg(hs1, src2d, dst2d, zeros2)
    s = _tc_post1(aggp[:, :N], hs1, dinv, b1.reshape(1, D), W2)
    s_pad = jnp.concatenate([s.reshape(N), jnp.zeros((NP_ - N,), jnp.float32)])
    qp = _sc_scalar_agg(s_pad, src2d, dst2d, zeros1)
    out = _tc_post2(qp[0, :N].reshape(N, 1), qp[1, :N].reshape(N, 1), s, dinv,
                    b2.reshape(1, 1))
    return out


# speedup vs baseline: 1.0009x; 1.0009x over previous
import functools

import jax
import jax.numpy as jnp
from jax import lax
from jax.experimental import pallas as pl
from jax.experimental.pallas import tpu as pltpu
from jax.experimental.pallas import tpu_sc as plsc

N = 10000
E = 320000
D = 128

NC = 2
NS = 16
NW = NC * NS

CHUNK = 128
CPT = 80
NCHUNKS = NW * CPT
E_PAD = NCHUNKS * CHUNK
NP_ = 10240
PAD_ROW = N
RPS = NP_ // NS

_MESH = plsc.VectorSubcoreMesh(core_axis_name="c", subcore_axis_name="s")


def _scalar_agg_body(gather, vals_hbm, src_hbm, dst_hbm, zeros_hbm, out_hbm,
                     sivb, divb, vbuf, acc, vsh, gsem, ssem):
    cid = lax.axis_index("c")
    sid = lax.axis_index("s")
    wid = sid * NC + cid
    c0 = wid * CPT
    r0 = sid * RPS
    pltpu.sync_copy(zeros_hbm.at[pl.ds(r0, RPS)], acc.at[pl.ds(r0, RPS)])
    pltpu.sync_copy(dst_hbm.at[pl.ds(c0, CPT)], divb)
    if gather:
        pltpu.sync_copy(src_hbm.at[pl.ds(c0, CPT)], sivb)
        pltpu.sync_copy(vals_hbm.at[pl.ds(r0, RPS)], vsh.at[pl.ds(r0, RPS)])
    else:
        for j in range(CHUNK // 16):
            vbuf[0, pl.ds(j * 16, 16)] = jnp.full((16,), 1.0, jnp.float32)
    plsc.subcore_barrier()

    def drain_g(b):
        pltpu.make_async_copy(vsh.at[sivb.at[0]], vbuf.at[b], gsem).wait()

    def drain_s(b):
        pltpu.make_async_copy(vbuf.at[b], acc.at[divb.at[0]], ssem).wait()

    if gather:
        pltpu.async_copy(vsh.at[sivb.at[0]], vbuf.at[0], gsem)

        def pair(i, carry):
            for b in (0, 1):
                c = 2 * i + b
                drain_g(b)
                pltpu.async_copy(vbuf.at[b], acc.at[divb.at[c]], ssem, add=True)

                @pl.when(c + 1 < CPT)
                def _():
                    @pl.when(c >= 1)
                    def _():
                        drain_s(1 - b)
                    pltpu.async_copy(vsh.at[sivb.at[c + 1]], vbuf.at[1 - b],
                                     gsem)

            return carry

        lax.fori_loop(0, CPT // 2, pair, 0)
        drain_s(0)
        drain_s(1)
    else:
        def body(c, carry):
            @pl.when(c >= 8)
            def _():
                drain_s(0)
            pltpu.async_copy(vbuf.at[0], acc.at[divb.at[c]], ssem, add=True)
            return carry

        lax.fori_loop(0, CPT, body, 0)
        for _ in range(8):
            drain_s(0)
    plsc.subcore_barrier()
    pltpu.sync_copy(acc.at[pl.ds(r0, RPS)], out_hbm.at[cid].at[pl.ds(r0, RPS)])


def _make_scalar_agg(gather):
    return functools.partial(
        pl.kernel,
        out_type=jax.ShapeDtypeStruct((NC, NP_), jnp.float32),
        mesh=_MESH,
        scratch_types=[
            pltpu.VMEM((CPT, CHUNK), jnp.int32),
            pltpu.VMEM((CPT, CHUNK), jnp.int32),
            pltpu.VMEM((2, CHUNK), jnp.float32),
            pltpu.VMEM_SHARED((NP_,), jnp.float32),
            pltpu.VMEM_SHARED((NP_,), jnp.float32),
            pltpu.SemaphoreType.DMA,
            pltpu.SemaphoreType.DMA,
        ],
    )(functools.partial(_scalar_agg_body, gather))


_sc_scalar_agg = _make_scalar_agg(True)
_sc_degree = _make_scalar_agg(False)


@functools.partial(
    pl.kernel,
    out_type=jax.ShapeDtypeStruct((NC, NP_, D), jnp.float32),
    mesh=_MESH,
    scratch_types=[
        pltpu.VMEM((CPT // 2, CHUNK), jnp.int32),
        pltpu.VMEM((CPT // 2, CHUNK), jnp.int32),
        pltpu.VMEM((2, CHUNK, D), jnp.float32),
        pltpu.VMEM_SHARED((NP_, D), jnp.float32),
        pltpu.SemaphoreType.DMA,
        pltpu.SemaphoreType.DMA,
    ],
)
def _sc_dense_agg(hs_hbm, src_hbm, dst_hbm, zeros_hbm, out_hbm,
                  sivb, divb, rows, acc, gsem, ssem):
    cid = lax.axis_index("c")
    sid = lax.axis_index("s")
    wid = sid * NC + cid
    c0 = wid * CPT
    r0 = sid * RPS
    HC = CPT // 2
    pltpu.sync_copy(zeros_hbm.at[pl.ds(r0, RPS)], acc.at[pl.ds(r0, RPS)])
    plsc.subcore_barrier()

    def drain_s(b):
        pltpu.make_async_copy(rows.at[b], acc.at[divb.at[0]], ssem).wait()

    for p in (0, 1):
        pltpu.sync_copy(src_hbm.at[pl.ds(c0 + p * HC, HC)], sivb)
        pltpu.sync_copy(dst_hbm.at[pl.ds(c0 + p * HC, HC)], divb)

        def pair(i, carry):
            for b in (0, 1):
                c = 2 * i + b

                @pl.when(i >= 1)
                def _():
                    drain_s(b)
                pltpu.sync_copy(hs_hbm.at[sivb.at[c]], rows.at[b])
                pltpu.async_copy(rows.at[b], acc.at[divb.at[c]], ssem,
                                 add=True)

            return carry

        lax.fori_loop(0, HC // 2, pair, 0)
        drain_s(0)
        drain_s(1)
    plsc.subcore_barrier()
    pltpu.sync_copy(acc.at[pl.ds(r0, RPS)], out_hbm.at[cid].at[pl.ds(r0, RPS)])


RB = 1000
GRID = N // RB


def _t0_body(x_ref, w_ref, h_ref):
    h_ref[...] = jnp.dot(x_ref[...], w_ref[...],
                         preferred_element_type=jnp.float32)


def _tc_matmul(x, W1):
    return pl.pallas_call(
        _t0_body,
        grid=(GRID,),
        in_specs=[
            pl.BlockSpec((RB, D), lambda i: (i, 0)),
            pl.BlockSpec((D, D), lambda i: (0, 0)),
        ],
        out_specs=pl.BlockSpec((RB, D), lambda i: (i, 0)),
        out_shape=jax.ShapeDtypeStruct((N, D), jnp.float32),
    )(x, W1)


def _t1_body(h_ref, dsum_ref, hs_ref, dinv_ref):
    dinv = lax.rsqrt(dsum_ref[...] + 1.0)
    hs_ref[...] = h_ref[...] * dinv
    dinv_ref[...] = dinv


def _tc_scale(h, degsum):
    return pl.pallas_call(
        _t1_body,
        grid=(GRID,),
        in_specs=[
            pl.BlockSpec((RB, D), lambda i: (i, 0)),
            pl.BlockSpec((RB, 1), lambda i: (i, 0)),
        ],
        out_specs=[
            pl.BlockSpec((RB, D), lambda i: (i, 0)),
            pl.BlockSpec((RB, 1), lambda i: (i, 0)),
        ],
        out_shape=[
            jax.ShapeDtypeStruct((N, D), jnp.float32),
            jax.ShapeDtypeStruct((N, 1), jnp.float32),
        ],
    )(h, degsum)


def _t2_body(a_ref, hs_ref, dinv_ref, b1_ref, w2_ref, s_ref):
    dinv = dinv_ref[...]
    o = (a_ref[0] + a_ref[1] + hs_ref[...]) * dinv + b1_ref[...]
    o = jnp.maximum(o, 0.0)
    s_ref[...] = jnp.dot(o, w2_ref[...], preferred_element_type=jnp.float32) * dinv


def _tc_post1(aggp, hs, dinv, b1, W2):
    return pl.pallas_call(
        _t2_body,
        grid=(GRID,),
        in_specs=[
            pl.BlockSpec((NC, RB, D), lambda i: (0, i, 0)),
            pl.BlockSpec((RB, D), lambda i: (i, 0)),
            pl.BlockSpec((RB, 1), lambda i: (i, 0)),
            pl.BlockSpec((1, D), lambda i: (0, 0)),
            pl.BlockSpec((D, 1), lambda i: (0, 0)),
        ],
        out_specs=pl.BlockSpec((RB, 1), lambda i: (i, 0)),
        out_shape=jax.ShapeDtypeStruct((N, 1), jnp.float32),
    )(aggp, hs, dinv, b1, W2)


def _t3_body(q0_ref, q1_ref, s_ref, dinv_ref, b2_ref, out_ref):
    pre = (q0_ref[...] + q1_ref[...] + s_ref[...]) * dinv_ref[...] + b2_ref[...]
    out_ref[...] = jax.nn.sigmoid(pre)


def _tc_post2(q0, q1, s, dinv, b2):
    return pl.pallas_call(
        _t3_body,
        out_shape=jax.ShapeDtypeStruct((N, 1), jnp.float32),
    )(q0, q1, s, dinv, b2)


def kernel(x, edge_index, W1, b1, W2, b2):
    npad = E_PAD - E
    src2d = jnp.concatenate(
        [edge_index[0].astype(jnp.int32),
         jnp.zeros((npad,), jnp.int32)]).reshape(NCHUNKS, CHUNK)
    pad_dst = PAD_ROW + jnp.arange(npad, dtype=jnp.int32) % (NP_ - N)
    dst2d = jnp.concatenate(
        [edge_index[1].astype(jnp.int32), pad_dst]).reshape(NCHUNKS, CHUNK)
    zeros1 = jnp.zeros((NP_,), jnp.float32)
    zeros2 = jnp.zeros((NP_, D), jnp.float32)

    h1 = _tc_matmul(x, W1)
    degp = _sc_degree(zeros1, src2d, dst2d, zeros1)
    degsum = (degp[0, :N] + degp[1, :N]).reshape(N, 1)
    hs1, dinv = _tc_scale(h1, degsum)
    aggp = _sc_dense_agg(hs1, src2d, dst2d, zeros2)
    s = _tc_post1(aggp[:, :N], hs1, dinv, b1.reshape(1, D), W2)
    s_pad = jnp.concatenate([s.reshape(N), jnp.zeros((NP_ - N,), jnp.float32)])
    qp = _sc_scalar_agg(s_pad, src2d, dst2d, zeros1)
    out = _tc_post2(qp[0, :N].reshape(N, 1), qp[1, :N].reshape(N, 1), s, dinv,
                    b2.reshape(1, 1))
    return out


# speedup vs baseline: 1.1002x; 1.0992x over previous
import functools

import jax
import jax.numpy as jnp
from jax import lax
from jax.experimental import pallas as pl
from jax.experimental.pallas import tpu as pltpu
from jax.experimental.pallas import tpu_sc as plsc

N = 10000
E = 320000
D = 128

NC = 2
NS = 16
NW = NC * NS

CHUNK = 128
CPT = 80
NCHUNKS = NW * CPT
E_PAD = NCHUNKS * CHUNK
NP_ = 10240
PAD_ROW = N
RPS = NP_ // NS

_MESH = plsc.VectorSubcoreMesh(core_axis_name="c", subcore_axis_name="s")


def _scalar_agg_body(gather, vals_hbm, src_hbm, dst_hbm, zeros_hbm, out_hbm,
                     sivb, divb, vbuf, acc, vsh, gsem, ssem):
    cid = lax.axis_index("c")
    sid = lax.axis_index("s")
    wid = sid * NC + cid
    c0 = wid * CPT
    r0 = sid * RPS
    pltpu.sync_copy(zeros_hbm.at[pl.ds(r0, RPS)], acc.at[pl.ds(r0, RPS)])
    pltpu.sync_copy(dst_hbm.at[pl.ds(c0, CPT)], divb)
    if gather:
        pltpu.sync_copy(src_hbm.at[pl.ds(c0, CPT)], sivb)
        pltpu.sync_copy(vals_hbm.at[pl.ds(r0, RPS)], vsh.at[pl.ds(r0, RPS)])
    else:
        for j in range(CHUNK // 16):
            vbuf[0, pl.ds(j * 16, 16)] = jnp.full((16,), 1.0, jnp.float32)
    plsc.subcore_barrier()

    def drain_g(b):
        pltpu.make_async_copy(vsh.at[sivb.at[0]], vbuf.at[b], gsem).wait()

    def drain_s(b):
        pltpu.make_async_copy(vbuf.at[b], acc.at[divb.at[0]], ssem).wait()

    if gather:
        pltpu.async_copy(vsh.at[sivb.at[0]], vbuf.at[0], gsem)

        def pair(i, carry):
            for b in (0, 1):
                c = 2 * i + b
                drain_g(b)
                pltpu.async_copy(vbuf.at[b], acc.at[divb.at[c]], ssem, add=True)

                @pl.when(c + 1 < CPT)
                def _():
                    @pl.when(c >= 1)
                    def _():
                        drain_s(1 - b)
                    pltpu.async_copy(vsh.at[sivb.at[c + 1]], vbuf.at[1 - b],
                                     gsem)

            return carry

        lax.fori_loop(0, CPT // 2, pair, 0)
        drain_s(0)
        drain_s(1)
    else:
        def body(c, carry):
            @pl.when(c >= 8)
            def _():
                drain_s(0)
            pltpu.async_copy(vbuf.at[0], acc.at[divb.at[c]], ssem, add=True)
            return carry

        lax.fori_loop(0, CPT, body, 0)
        for _ in range(8):
            drain_s(0)
    plsc.subcore_barrier()
    pltpu.sync_copy(acc.at[pl.ds(r0, RPS)], out_hbm.at[cid].at[pl.ds(r0, RPS)])


def _make_scalar_agg(gather):
    return functools.partial(
        pl.kernel,
        out_type=jax.ShapeDtypeStruct((NC, NP_), jnp.float32),
        mesh=_MESH,
        scratch_types=[
            pltpu.VMEM((CPT, CHUNK), jnp.int32),
            pltpu.VMEM((CPT, CHUNK), jnp.int32),
            pltpu.VMEM((2, CHUNK), jnp.float32),
            pltpu.VMEM_SHARED((NP_,), jnp.float32),
            pltpu.VMEM_SHARED((NP_,), jnp.float32),
            pltpu.SemaphoreType.DMA,
            pltpu.SemaphoreType.DMA,
        ],
    )(functools.partial(_scalar_agg_body, gather))


_sc_scalar_agg = _make_scalar_agg(True)
_sc_degree = _make_scalar_agg(False)


CA = 128
CB = 32
PA = CA // 4
PB = CB // 2


@functools.partial(
    pl.kernel,
    out_type=jax.ShapeDtypeStruct((NC, NP_, D), jnp.float32),
    mesh=_MESH,
    scratch_types=[
        pltpu.VMEM((PA, CHUNK), jnp.int32),
        pltpu.VMEM((PA, CHUNK), jnp.int32),
        pltpu.VMEM((2, CHUNK, D), jnp.float32),
        pltpu.VMEM_SHARED((NP_, D), jnp.float32),
        pltpu.SemaphoreType.DMA,
        pltpu.SemaphoreType.DMA,
    ],
)
def _sc_dense_agg(hs_hbm, src_hbm, dst_hbm, zeros_hbm, out_hbm,
                  sivb, divb, rows, acc, gsem, ssem):
    cid = lax.axis_index("c")
    sid = lax.axis_index("s")
    r0 = sid * RPS
    pltpu.sync_copy(zeros_hbm.at[pl.ds(r0, RPS)], acc.at[pl.ds(r0, RPS)])
    plsc.subcore_barrier()

    def drain_s(b):
        pltpu.make_async_copy(rows.at[b], acc.at[divb.at[0]], ssem).wait()

    def run(c0, HC, nph):
        for p in range(nph):
            pltpu.sync_copy(src_hbm.at[pl.ds(c0 + p * HC, HC)],
                            sivb.at[pl.ds(0, HC)])
            pltpu.sync_copy(dst_hbm.at[pl.ds(c0 + p * HC, HC)],
                            divb.at[pl.ds(0, HC)])

            def pair(i, carry):
                for b in (0, 1):
                    c = 2 * i + b

                    @pl.when(i >= 1)
                    def _():
                        drain_s(b)
                    pltpu.sync_copy(hs_hbm.at[sivb.at[c]], rows.at[b])
                    pltpu.async_copy(rows.at[b], acc.at[divb.at[c]], ssem,
                                     add=True)

                return carry

            lax.fori_loop(0, HC // 2, pair, 0)
            drain_s(0)
            drain_s(1)

    @pl.when(cid == 0)
    def _():
        run(sid * CA, PA, 4)

    @pl.when(cid == 1)
    def _():
        run(NS * CA + sid * CB, PB, 2)

    plsc.subcore_barrier()
    pltpu.sync_copy(acc.at[pl.ds(r0, RPS)], out_hbm.at[cid].at[pl.ds(r0, RPS)])


RB = 1000
GRID = N // RB


def _t0_body(x_ref, w_ref, h_ref):
    h_ref[...] = jnp.dot(x_ref[...], w_ref[...],
                         preferred_element_type=jnp.float32)


def _tc_matmul(x, W1):
    return pl.pallas_call(
        _t0_body,
        grid=(GRID,),
        in_specs=[
            pl.BlockSpec((RB, D), lambda i: (i, 0)),
            pl.BlockSpec((D, D), lambda i: (0, 0)),
        ],
        out_specs=pl.BlockSpec((RB, D), lambda i: (i, 0)),
        out_shape=jax.ShapeDtypeStruct((N, D), jnp.float32),
    )(x, W1)


def _t1_body(h_ref, dsum_ref, hs_ref, dinv_ref):
    dinv = lax.rsqrt(dsum_ref[...] + 1.0)
    hs_ref[...] = h_ref[...] * dinv
    dinv_ref[...] = dinv


def _tc_scale(h, degsum):
    return pl.pallas_call(
        _t1_body,
        grid=(GRID,),
        in_specs=[
            pl.BlockSpec((RB, D), lambda i: (i, 0)),
            pl.BlockSpec((RB, 1), lambda i: (i, 0)),
        ],
        out_specs=[
            pl.BlockSpec((RB, D), lambda i: (i, 0)),
            pl.BlockSpec((RB, 1), lambda i: (i, 0)),
        ],
        out_shape=[
            jax.ShapeDtypeStruct((N, D), jnp.float32),
            jax.ShapeDtypeStruct((N, 1), jnp.float32),
        ],
    )(h, degsum)


def _t2_body(a_ref, hs_ref, dinv_ref, b1_ref, w2_ref, s_ref):
    dinv = dinv_ref[...]
    o = (a_ref[0] + a_ref[1] + hs_ref[...]) * dinv + b1_ref[...]
    o = jnp.maximum(o, 0.0)
    s_ref[...] = jnp.dot(o, w2_ref[...], preferred_element_type=jnp.float32) * dinv


def _tc_post1(aggp, hs, dinv, b1, W2):
    return pl.pallas_call(
        _t2_body,
        grid=(GRID,),
        in_specs=[
            pl.BlockSpec((NC, RB, D), lambda i: (0, i, 0)),
            pl.BlockSpec((RB, D), lambda i: (i, 0)),
            pl.BlockSpec((RB, 1), lambda i: (i, 0)),
            pl.BlockSpec((1, D), lambda i: (0, 0)),
            pl.BlockSpec((D, 1), lambda i: (0, 0)),
        ],
        out_specs=pl.BlockSpec((RB, 1), lambda i: (i, 0)),
        out_shape=jax.ShapeDtypeStruct((N, 1), jnp.float32),
    )(aggp, hs, dinv, b1, W2)


def _t3_body(q0_ref, q1_ref, s_ref, dinv_ref, b2_ref, out_ref):
    pre = (q0_ref[...] + q1_ref[...] + s_ref[...]) * dinv_ref[...] + b2_ref[...]
    out_ref[...] = jax.nn.sigmoid(pre)


def _tc_post2(q0, q1, s, dinv, b2):
    return pl.pallas_call(
        _t3_body,
        out_shape=jax.ShapeDtypeStruct((N, 1), jnp.float32),
    )(q0, q1, s, dinv, b2)


def kernel(x, edge_index, W1, b1, W2, b2):
    npad = E_PAD - E
    src2d = jnp.concatenate(
        [edge_index[0].astype(jnp.int32),
         jnp.zeros((npad,), jnp.int32)]).reshape(NCHUNKS, CHUNK)
    pad_dst = PAD_ROW + jnp.arange(npad, dtype=jnp.int32) % (NP_ - N)
    dst2d = jnp.concatenate(
        [edge_index[1].astype(jnp.int32), pad_dst]).reshape(NCHUNKS, CHUNK)
    zeros1 = jnp.zeros((NP_,), jnp.float32)
    zeros2 = jnp.zeros((NP_, D), jnp.float32)

    h1 = _tc_matmul(x, W1)
    degp = _sc_degree(zeros1, src2d, dst2d, zeros1)
    degsum = (degp[0, :N] + degp[1, :N]).reshape(N, 1)
    hs1, dinv = _tc_scale(h1, degsum)
    aggp = _sc_dense_agg(hs1, src2d, dst2d, zeros2)
    s = _tc_post1(aggp[:, :N], hs1, dinv, b1.reshape(1, D), W2)
    s_pad = jnp.concatenate([s.reshape(N), jnp.zeros((NP_ - N,), jnp.float32)])
    qp = _sc_scalar_agg(s_pad, src2d, dst2d, zeros1)
    out = _tc_post2(qp[0, :N].reshape(N, 1), qp[1, :N].reshape(N, 1), s, dinv,
                    b2.reshape(1, 1))
    return out


# speedup vs baseline: 2.5324x; 2.3018x over previous
import functools

import jax
import jax.numpy as jnp
from jax import lax
from jax.experimental import pallas as pl
from jax.experimental.pallas import tpu as pltpu
from jax.experimental.pallas import tpu_sc as plsc

N = 10000
E = 320000
D = 128

NC = 2
NS = 16
NW = NC * NS

CHUNK = 128
CPT = 80
NCHUNKS = NW * CPT
E_PAD = NCHUNKS * CHUNK
NP_ = 10240
PAD_ROW = N
RPS = NP_ // NS

_MESH = plsc.VectorSubcoreMesh(core_axis_name="c", subcore_axis_name="s")


def _scalar_agg_body(gather, vals_hbm, src_hbm, dst_hbm, zeros_hbm, out_hbm,
                     sivb, divb, vbuf, acc, vsh, gsem, ssem):
    cid = lax.axis_index("c")
    sid = lax.axis_index("s")
    wid = sid * NC + cid
    c0 = wid * CPT
    r0 = sid * RPS
    pltpu.sync_copy(zeros_hbm.at[pl.ds(r0, RPS)], acc.at[pl.ds(r0, RPS)])
    pltpu.sync_copy(dst_hbm.at[pl.ds(c0, CPT)], divb)
    if gather:
        pltpu.sync_copy(src_hbm.at[pl.ds(c0, CPT)], sivb)
        pltpu.sync_copy(vals_hbm.at[pl.ds(r0, RPS)], vsh.at[pl.ds(r0, RPS)])
    else:
        for j in range(CHUNK // 16):
            vbuf[0, pl.ds(j * 16, 16)] = jnp.full((16,), 1.0, jnp.float32)
    plsc.subcore_barrier()

    def drain_g(b):
        pltpu.make_async_copy(vsh.at[sivb.at[0]], vbuf.at[b], gsem).wait()

    def drain_s(b):
        pltpu.make_async_copy(vbuf.at[b], acc.at[divb.at[0]], ssem).wait()

    if gather:
        pltpu.async_copy(vsh.at[sivb.at[0]], vbuf.at[0], gsem)

        def pair(i, carry):
            for b in (0, 1):
                c = 2 * i + b
                drain_g(b)
                pltpu.async_copy(vbuf.at[b], acc.at[divb.at[c]], ssem, add=True)

                @pl.when(c + 1 < CPT)
                def _():
                    @pl.when(c >= 1)
                    def _():
                        drain_s(1 - b)
                    pltpu.async_copy(vsh.at[sivb.at[c + 1]], vbuf.at[1 - b],
                                     gsem)

            return carry

        lax.fori_loop(0, CPT // 2, pair, 0)
        drain_s(0)
        drain_s(1)
    else:
        def body(c, carry):
            @pl.when(c >= 8)
            def _():
                drain_s(0)
            pltpu.async_copy(vbuf.at[0], acc.at[divb.at[c]], ssem, add=True)
            return carry

        lax.fori_loop(0, CPT, body, 0)
        for _ in range(8):
            drain_s(0)
    plsc.subcore_barrier()
    pltpu.sync_copy(acc.at[pl.ds(r0, RPS)], out_hbm.at[cid].at[pl.ds(r0, RPS)])


def _make_scalar_agg(gather):
    return functools.partial(
        pl.kernel,
        out_type=jax.ShapeDtypeStruct((NC, NP_), jnp.float32),
        mesh=_MESH,
        scratch_types=[
            pltpu.VMEM((CPT, CHUNK), jnp.int32),
            pltpu.VMEM((CPT, CHUNK), jnp.int32),
            pltpu.VMEM((2, CHUNK), jnp.float32),
            pltpu.VMEM_SHARED((NP_,), jnp.float32),
            pltpu.VMEM_SHARED((NP_,), jnp.float32),
            pltpu.SemaphoreType.DMA,
            pltpu.SemaphoreType.DMA,
        ],
    )(functools.partial(_scalar_agg_body, gather))


_sc_scalar_agg = _make_scalar_agg(True)
_sc_degree = _make_scalar_agg(False)


CA = 80
CB = 80
PA = CA // 2
PB = CB // 2


@functools.partial(
    pl.kernel,
    out_type=jax.ShapeDtypeStruct((NC, NP_, D), jnp.float32),
    mesh=_MESH,
    scratch_types=[
        pltpu.VMEM((PA, CHUNK), jnp.int32),
        pltpu.VMEM((PA, CHUNK), jnp.int32),
        pltpu.VMEM((2, CHUNK, D), jnp.float32),
        pltpu.VMEM_SHARED((NP_, D), jnp.float32),
        pltpu.SemaphoreType.DMA,
        pltpu.SemaphoreType.DMA,
    ],
)
def _sc_dense_agg(hs_hbm, src_hbm, dst_hbm, zeros_hbm, out_hbm,
                  sivb, divb, rows, acc, gsem, ssem):
    cid = lax.axis_index("c")
    sid = lax.axis_index("s")
    r0 = sid * RPS
    pltpu.sync_copy(zeros_hbm.at[pl.ds(r0, RPS)], acc.at[pl.ds(r0, RPS)])
    plsc.subcore_barrier()

    def drain_s(b):
        pltpu.make_async_copy(rows.at[b], acc.at[divb.at[0]], ssem).wait()

    def run(c0, HC, nph):
        for p in range(nph):
            pltpu.sync_copy(src_hbm.at[pl.ds(c0 + p * HC, HC)],
                            sivb.at[pl.ds(0, HC)])
            pltpu.sync_copy(dst_hbm.at[pl.ds(c0 + p * HC, HC)],
                            divb.at[pl.ds(0, HC)])

            def pair(i, carry):
                for b in (0, 1):
                    c = 2 * i + b

                    @pl.when(i >= 1)
                    def _():
                        drain_s(b)
                    pltpu.sync_copy(hs_hbm.at[sivb.at[c]], rows.at[b])
                    pltpu.async_copy(rows.at[b], acc.at[divb.at[c]], ssem,
                                     add=True)

                return carry

            lax.fori_loop(0, HC // 2, pair, 0)
            drain_s(0)
            drain_s(1)

    @pl.when(cid == 0)
    def _():
        run(sid * CA, PA, 2)

    @pl.when(cid == 1)
    def _():
        run(NS * CA + sid * CB, PB, 2)

    plsc.subcore_barrier()
    pltpu.sync_copy(acc.at[pl.ds(r0, RPS)], out_hbm.at[cid].at[pl.ds(r0, RPS)])


RB = 1000
GRID = N // RB


def _t0_body(x_ref, w_ref, h_ref):
    h_ref[...] = jnp.dot(x_ref[...], w_ref[...],
                         preferred_element_type=jnp.float32)


def _tc_matmul(x, W1):
    return pl.pallas_call(
        _t0_body,
        grid=(GRID,),
        in_specs=[
            pl.BlockSpec((RB, D), lambda i: (i, 0)),
            pl.BlockSpec((D, D), lambda i: (0, 0)),
        ],
        out_specs=pl.BlockSpec((RB, D), lambda i: (i, 0)),
        out_shape=jax.ShapeDtypeStruct((N, D), jnp.float32),
    )(x, W1)


def _t1_body(h_ref, dsum_ref, hs_ref, dinv_ref):
    dinv = lax.rsqrt(dsum_ref[...] + 1.0)
    hs_ref[...] = h_ref[...] * dinv
    dinv_ref[...] = dinv


def _tc_scale(h, degsum):
    return pl.pallas_call(
        _t1_body,
        grid=(GRID,),
        in_specs=[
            pl.BlockSpec((RB, D), lambda i: (i, 0)),
            pl.BlockSpec((RB, 1), lambda i: (i, 0)),
        ],
        out_specs=[
            pl.BlockSpec((RB, D), lambda i: (i, 0)),
            pl.BlockSpec((RB, 1), lambda i: (i, 0)),
        ],
        out_shape=[
            jax.ShapeDtypeStruct((N, D), jnp.float32),
            jax.ShapeDtypeStruct((N, 1), jnp.float32),
        ],
    )(h, degsum)


def _t2_body(a_ref, hs_ref, dinv_ref, b1_ref, w2_ref, s_ref):
    dinv = dinv_ref[...]
    o = (a_ref[0] + a_ref[1] + hs_ref[...]) * dinv + b1_ref[...]
    o = jnp.maximum(o, 0.0)
    s_ref[...] = jnp.dot(o, w2_ref[...], preferred_element_type=jnp.float32) * dinv


def _tc_post1(aggp, hs, dinv, b1, W2):
    return pl.pallas_call(
        _t2_body,
        grid=(GRID,),
        in_specs=[
            pl.BlockSpec((NC, RB, D), lambda i: (0, i, 0)),
            pl.BlockSpec((RB, D), lambda i: (i, 0)),
            pl.BlockSpec((RB, 1), lambda i: (i, 0)),
            pl.BlockSpec((1, D), lambda i: (0, 0)),
            pl.BlockSpec((D, 1), lambda i: (0, 0)),
        ],
        out_specs=pl.BlockSpec((RB, 1), lambda i: (i, 0)),
        out_shape=jax.ShapeDtypeStruct((N, 1), jnp.float32),
    )(aggp, hs, dinv, b1, W2)


def _t3_body(q0_ref, q1_ref, s_ref, dinv_ref, b2_ref, out_ref):
    pre = (q0_ref[...] + q1_ref[...] + s_ref[...]) * dinv_ref[...] + b2_ref[...]
    out_ref[...] = jax.nn.sigmoid(pre)


def _tc_post2(q0, q1, s, dinv, b2):
    return pl.pallas_call(
        _t3_body,
        out_shape=jax.ShapeDtypeStruct((N, 1), jnp.float32),
    )(q0, q1, s, dinv, b2)


def kernel(x, edge_index, W1, b1, W2, b2):
    npad = E_PAD - E
    pad_src = jnp.arange(npad, dtype=jnp.int32) % N
    src2d = jnp.concatenate(
        [edge_index[0].astype(jnp.int32), pad_src]).reshape(NCHUNKS, CHUNK)
    pad_dst = PAD_ROW + jnp.arange(npad, dtype=jnp.int32) % (NP_ - N)
    dst2d = jnp.concatenate(
        [edge_index[1].astype(jnp.int32), pad_dst]).reshape(NCHUNKS, CHUNK)
    zeros1 = jnp.zeros((NP_,), jnp.float32)
    zeros2 = jnp.zeros((NP_, D), jnp.float32)

    h1 = _tc_matmul(x, W1)
    degp = _sc_degree(zeros1, src2d, dst2d, zeros1)
    degsum = (degp[0, :N] + degp[1, :N]).reshape(N, 1)
    hs1, dinv = _tc_scale(h1, degsum)
    aggp = _sc_dense_agg(hs1, src2d, dst2d, zeros2)
    s = _tc_post1(aggp[:, :N], hs1, dinv, b1.reshape(1, D), W2)
    s_pad = jnp.concatenate([s.reshape(N), jnp.zeros((NP_ - N,), jnp.float32)])
    qp = _sc_scalar_agg(s_pad, src2d, dst2d, zeros1)
    out = _tc_post2(qp[0, :N].reshape(N, 1), qp[1, :N].reshape(N, 1), s, dinv,
                    b2.reshape(1, 1))
    return out


# speedup vs baseline: 2.5789x; 1.0184x over previous
import functools

import jax
import jax.numpy as jnp
from jax import lax
from jax.experimental import pallas as pl
from jax.experimental.pallas import tpu as pltpu
from jax.experimental.pallas import tpu_sc as plsc

N = 10000
E = 320000
D = 128

NC = 2
NS = 16
NW = NC * NS

CHUNK = 128
CPT = 80
NCHUNKS = NW * CPT
E_PAD = NCHUNKS * CHUNK
NP_ = 10240
PAD_ROW = N
RPS = NP_ // NS

_MESH = plsc.VectorSubcoreMesh(core_axis_name="c", subcore_axis_name="s")


def _scalar_agg_body(gather, vals_hbm, src_hbm, dst_hbm, zeros_hbm, out_hbm,
                     sivb, divb, vbuf, acc, vsh, gsem, ssem):
    cid = lax.axis_index("c")
    sid = lax.axis_index("s")
    wid = sid * NC + cid
    c0 = wid * CPT
    r0 = sid * RPS
    pltpu.sync_copy(zeros_hbm.at[pl.ds(r0, RPS)], acc.at[pl.ds(r0, RPS)])
    pltpu.sync_copy(dst_hbm.at[pl.ds(c0, CPT)], divb)
    if gather:
        pltpu.sync_copy(src_hbm.at[pl.ds(c0, CPT)], sivb)
        pltpu.sync_copy(vals_hbm.at[pl.ds(r0, RPS)], vsh.at[pl.ds(r0, RPS)])
    else:
        for j in range(CHUNK // 16):
            vbuf[0, pl.ds(j * 16, 16)] = jnp.full((16,), 1.0, jnp.float32)
    plsc.subcore_barrier()

    def drain_g(b):
        pltpu.make_async_copy(vsh.at[sivb.at[0]], vbuf.at[b], gsem).wait()

    def drain_s(b):
        pltpu.make_async_copy(vbuf.at[b], acc.at[divb.at[0]], ssem).wait()

    if gather:
        pltpu.async_copy(vsh.at[sivb.at[0]], vbuf.at[0], gsem)

        def pair(i, carry):
            for b in (0, 1):
                c = 2 * i + b
                drain_g(b)
                pltpu.async_copy(vbuf.at[b], acc.at[divb.at[c]], ssem, add=True)

                @pl.when(c + 1 < CPT)
                def _():
                    @pl.when(c >= 1)
                    def _():
                        drain_s(1 - b)
                    pltpu.async_copy(vsh.at[sivb.at[c + 1]], vbuf.at[1 - b],
                                     gsem)

            return carry

        lax.fori_loop(0, CPT // 2, pair, 0)
        drain_s(0)
        drain_s(1)
    else:
        def body(c, carry):
            @pl.when(c >= 8)
            def _():
                drain_s(0)
            pltpu.async_copy(vbuf.at[0], acc.at[divb.at[c]], ssem, add=True)
            return carry

        lax.fori_loop(0, CPT, body, 0)
        for _ in range(8):
            drain_s(0)
    plsc.subcore_barrier()
    pltpu.sync_copy(acc.at[pl.ds(r0, RPS)], out_hbm.at[cid].at[pl.ds(r0, RPS)])


def _make_scalar_agg(gather):
    return functools.partial(
        pl.kernel,
        out_type=jax.ShapeDtypeStruct((NC, NP_), jnp.float32),
        mesh=_MESH,
        scratch_types=[
            pltpu.VMEM((CPT, CHUNK), jnp.int32),
            pltpu.VMEM((CPT, CHUNK), jnp.int32),
            pltpu.VMEM((2, CHUNK), jnp.float32),
            pltpu.VMEM_SHARED((NP_,), jnp.float32),
            pltpu.VMEM_SHARED((NP_,), jnp.float32),
            pltpu.SemaphoreType.DMA,
            pltpu.SemaphoreType.DMA,
        ],
    )(functools.partial(_scalar_agg_body, gather))


_sc_scalar_agg = _make_scalar_agg(True)
_sc_degree = _make_scalar_agg(False)


CA = 80
CB = 80
PA = CA // 2
PB = CB // 2


@functools.partial(
    pl.kernel,
    out_type=jax.ShapeDtypeStruct((NC, NP_, D), jnp.float32),
    mesh=_MESH,
    scratch_types=[
        pltpu.VMEM((PA, CHUNK), jnp.int32),
        pltpu.VMEM((PA, CHUNK), jnp.int32),
        pltpu.VMEM((2, CHUNK, D), jnp.float32),
        pltpu.VMEM_SHARED((NP_, D), jnp.float32),
        pltpu.SemaphoreType.DMA,
        pltpu.SemaphoreType.DMA,
    ],
)
def _sc_dense_agg(hs_hbm, src_hbm, dst_hbm, zeros_hbm, out_hbm,
                  sivb, divb, rows, acc, gsem, ssem):
    cid = lax.axis_index("c")
    sid = lax.axis_index("s")
    r0 = sid * RPS
    pltpu.sync_copy(zeros_hbm.at[pl.ds(r0, RPS)], acc.at[pl.ds(r0, RPS)])
    plsc.subcore_barrier()

    def drain_s(b):
        pltpu.make_async_copy(rows.at[b], acc.at[divb.at[0]], ssem).wait()

    def run(c0, HC, nph):
        for p in range(nph):
            pltpu.sync_copy(src_hbm.at[pl.ds(c0 + p * HC, HC)],
                            sivb.at[pl.ds(0, HC)])
            pltpu.sync_copy(dst_hbm.at[pl.ds(c0 + p * HC, HC)],
                            divb.at[pl.ds(0, HC)])

            def pair(i, carry):
                for b in (0, 1):
                    c = 2 * i + b

                    @pl.when(i >= 1)
                    def _():
                        drain_s(b)
                    pltpu.sync_copy(hs_hbm.at[sivb.at[c]], rows.at[b])
                    pltpu.async_copy(rows.at[b], acc.at[divb.at[c]], ssem,
                                     add=True)

                return carry

            lax.fori_loop(0, HC // 2, pair, 0)
            drain_s(0)
            drain_s(1)

    @pl.when(cid == 0)
    def _():
        run(sid * CA, PA, 2)

    @pl.when(cid == 1)
    def _():
        run(NS * CA + sid * CB, PB, 2)

    plsc.subcore_barrier()
    pltpu.sync_copy(acc.at[pl.ds(r0, RPS)], out_hbm.at[cid].at[pl.ds(r0, RPS)])


RB = 1000
GRID = N // RB


def _t0_body(x_ref, w_ref, h_ref):
    h_ref[...] = jnp.dot(x_ref[...], w_ref[...],
                         preferred_element_type=jnp.float32)


def _tc_matmul(x, W1):
    return pl.pallas_call(
        _t0_body,
        grid=(GRID,),
        in_specs=[
            pl.BlockSpec((RB, D), lambda i: (i, 0)),
            pl.BlockSpec((D, D), lambda i: (0, 0)),
        ],
        out_specs=pl.BlockSpec((RB, D), lambda i: (i, 0)),
        out_shape=jax.ShapeDtypeStruct((N, D), jnp.float32),
    )(x, W1)


def _t1_body(h_ref, degp_ref, hs_ref, dinv_ref):
    deg = degp_ref[0] + degp_ref[1]
    dinv = lax.rsqrt(deg + 1.0)
    hs_ref[...] = h_ref[...] * dinv
    dinv_ref[...] = dinv


def _tc_scale(h, degp):
    return pl.pallas_call(
        _t1_body,
        grid=(GRID,),
        in_specs=[
            pl.BlockSpec((RB, D), lambda i: (i, 0)),
            pl.BlockSpec((NC, RB, 1), lambda i: (0, i, 0)),
        ],
        out_specs=[
            pl.BlockSpec((RB, D), lambda i: (i, 0)),
            pl.BlockSpec((RB, 1), lambda i: (i, 0)),
        ],
        out_shape=[
            jax.ShapeDtypeStruct((N, D), jnp.float32),
            jax.ShapeDtypeStruct((N, 1), jnp.float32),
        ],
    )(h, degp.reshape(NC, NP_, 1))


def _t2_body(a_ref, hs_ref, dinv_ref, b1_ref, w2_ref, s_ref):
    dinv = dinv_ref[...]
    o = (a_ref[0] + a_ref[1] + hs_ref[...]) * dinv + b1_ref[...]
    o = jnp.maximum(o, 0.0)
    s_ref[...] = jnp.dot(o, w2_ref[...], preferred_element_type=jnp.float32) * dinv


def _tc_post1(aggp, hs, dinv, b1, W2):
    return pl.pallas_call(
        _t2_body,
        grid=(GRID,),
        in_specs=[
            pl.BlockSpec((NC, RB, D), lambda i: (0, i, 0)),
            pl.BlockSpec((RB, D), lambda i: (i, 0)),
            pl.BlockSpec((RB, 1), lambda i: (i, 0)),
            pl.BlockSpec((1, D), lambda i: (0, 0)),
            pl.BlockSpec((D, 1), lambda i: (0, 0)),
        ],
        out_specs=pl.BlockSpec((RB, 1), lambda i: (i, 0)),
        out_shape=jax.ShapeDtypeStruct((NP_, 1), jnp.float32),
    )(aggp, hs, dinv, b1, W2)


def _t3_body(qp_ref, s_ref, dinv_ref, b2_ref, out_ref):
    q = qp_ref[0] + qp_ref[1]
    pre = (q + s_ref[...]) * dinv_ref[...] + b2_ref[...]
    out_ref[...] = jax.nn.sigmoid(pre)


def _tc_post2(qp, s2, dinv, b2):
    return pl.pallas_call(
        _t3_body,
        grid=(GRID,),
        in_specs=[
            pl.BlockSpec((NC, RB, 1), lambda i: (0, i, 0)),
            pl.BlockSpec((RB, 1), lambda i: (i, 0)),
            pl.BlockSpec((RB, 1), lambda i: (i, 0)),
            pl.BlockSpec((1, 1), lambda i: (0, 0)),
        ],
        out_specs=pl.BlockSpec((RB, 1), lambda i: (i, 0)),
        out_shape=jax.ShapeDtypeStruct((N, 1), jnp.float32),
    )(qp.reshape(NC, NP_, 1), s2, dinv, b2)


def kernel(x, edge_index, W1, b1, W2, b2):
    npad = E_PAD - E
    pad_src = jnp.arange(npad, dtype=jnp.int32) % N
    src2d = jnp.concatenate(
        [edge_index[0].astype(jnp.int32), pad_src]).reshape(NCHUNKS, CHUNK)
    pad_dst = PAD_ROW + jnp.arange(npad, dtype=jnp.int32) % (NP_ - N)
    dst2d = jnp.concatenate(
        [edge_index[1].astype(jnp.int32), pad_dst]).reshape(NCHUNKS, CHUNK)
    zeros1 = jnp.zeros((NP_,), jnp.float32)
    zeros2 = jnp.zeros((NP_, D), jnp.float32)

    h1 = _tc_matmul(x, W1)
    degp = _sc_degree(zeros1, src2d, dst2d, zeros1)
    hs1, dinv = _tc_scale(h1, degp)
    aggp = _sc_dense_agg(hs1, src2d, dst2d, zeros2)
    s2 = _tc_post1(aggp, hs1, dinv, b1.reshape(1, D), W2)
    qp = _sc_scalar_agg(s2.reshape(NP_), src2d, dst2d, zeros1)
    out = _tc_post2(qp, s2, dinv, b2.reshape(1, 1))
    return out
